# baseline (speedup 1.0000x reference)
import functools

import jax
import jax.numpy as jnp
from jax import lax
from jax.experimental import pallas as pl
from jax.experimental.pallas import tpu as pltpu
from jax.experimental.pallas import tpu_sc as plsc

N = 10000
IN = 128
HID = 16
HEADS = 8
OUT = 128
E = 320000

NP = 10240
NPD = NP + NP // 8
NC = 2
NS = 16
NW = NC * NS
C = 16
STEPS = 646
EP = NW * C * STEPS
RPD = NPD // NS


def _edge_pass(e_pack, h, t_tab):
    mesh = plsc.VectorSubcoreMesh(core_axis_name="c", subcore_axis_name="s",
                                  num_cores=NC, num_subcores=NS)

    @functools.partial(
        pl.kernel,
        out_type=jax.ShapeDtypeStruct((NC, NPD, 128), jnp.float32),
        mesh=mesh,
        scratch_types=[
            pltpu.VMEM((STEPS * 2 * C,), jnp.int32),
            pltpu.VMEM((2 * C, 128), jnp.float32),
            pltpu.VMEM((2 * C, 128), jnp.float32),
            pltpu.VMEM((2 * C, 128), jnp.float32),
            pltpu.VMEM((2 * C, 128), jnp.float32),
            pltpu.VMEM((2 * C,), jnp.int32),
            pltpu.VMEM((2 * C,), jnp.int32),
            pltpu.VMEM_SHARED((NPD, 128), jnp.float32),
            pltpu.SemaphoreType.DMA,
            pltpu.SemaphoreType.DMA,
            pltpu.SemaphoreType.DMA,
            pltpu.SemaphoreType.DMA,
            pltpu.SemaphoreType.DMA,
            pltpu.SemaphoreType.DMA,
        ],
    )
    def k(e_h, h_h, t_h, acc_o, eidx_v, comb_a, comb_b, g_a, g_b,
          sci_a, sci_b, acc_sh, s0a, s1a, s0b, s1b, s2a, s2b):
        cid = lax.axis_index("c")
        sid = lax.axis_index("s")
        wid = cid * NS + sid
        r0 = sid * RPD

        pltpu.sync_copy(e_h.at[pl.ds(wid * STEPS * 2 * C, STEPS * 2 * C)],
                        eidx_v)

        for comb_v in (comb_a, comb_b):
            @pl.loop(0, 2 * C)
            def _(e, comb_v=comb_v):
                for c in range(8):
                    comb_v[e, pl.ds(16 * c, 16)] = jnp.zeros((16,),
                                                             jnp.float32)

        for p in range(RPD // 40):
            pltpu.sync_copy(comb_a.at[pl.ds(0, 40)],
                            acc_sh.at[pl.ds(r0 + p * 40, 40)])
        plsc.subcore_barrier()

        def launch(st, g_v, comb_v, s0, s1):
            ib = st * 2 * C
            cg = pltpu.async_copy(t_h.at[eidx_v.at[pl.ds(ib, 2 * C)]], g_v, s0)
            ch = pltpu.async_copy(h_h.at[eidx_v.at[pl.ds(ib, C)]],
                                  comb_v.at[pl.ds(0, C)], s1)
            return cg, ch

        def compute(st, g_v, comb_v, scidx_v, cg, ch):
            ib = st * 2 * C
            cg.wait()
            ch.wait()

            @pl.loop(0, C // 16)
            def _(j):
                d16 = eidx_v[pl.ds(ib + C + 16 * j, 16)]
                scidx_v[pl.ds(16 * j, 16)] = d16
                scidx_v[pl.ds(C + 16 * j, 16)] = (
                    NP + lax.shift_right_logical(d16, 3))
                slot16 = (d16 & 7) * 16
                for i in range(16):
                    e = 16 * j + i
                    t = g_v[e, pl.ds(0, 16)] + g_v[C + e, pl.ds(16, 16)]
                    t = jnp.where(t >= 0.0, t, 0.2 * t)
                    w = jnp.exp(t)
                    comb_v[C + e, pl.ds(slot16[i], 16)] = w
                    for m in range(8):
                        comb_v[e, pl.ds(16 * m, 16)] = (
                            comb_v[e, pl.ds(16 * m, 16)] * w[m])

        def zero_restore(st, comb_v):
            ib = st * 2 * C

            @pl.loop(0, C // 16)
            def _(j):
                d16 = eidx_v[pl.ds(ib + C + 16 * j, 16)]
                slot16 = (d16 & 7) * 16
                for i in range(16):
                    comb_v[C + 16 * j + i, pl.ds(slot16[i], 16)] = (
                        jnp.zeros((16,), jnp.float32))

        prime = launch(0, g_a, comb_a, s0a, s1a)
        scb_drain = pltpu.make_async_copy(comb_b, acc_sh.at[sci_b], s2b)

        @pl.loop(0, STEPS, step=2)
        def _(st):
            @pl.when(st >= 2)
            def _():
                scb_drain.wait()
                zero_restore(st - 1, comb_b)

            cgb, chb = launch(st + 1, g_b, comb_b, s0b, s1b)
            compute(st, g_a, comb_a, sci_a, *prime)
            sca = pltpu.async_copy(comb_a, acc_sh.at[sci_a], s2a, add=True)
            compute(st + 1, g_b, comb_b, sci_b, cgb, chb)
            pltpu.async_copy(comb_b, acc_sh.at[sci_b], s2b, add=True)
            sca.wait()
            zero_restore(st, comb_a)

            @pl.when(st + 2 < STEPS)
            def _():
                launch(st + 2, g_a, comb_a, s0a, s1a)

        scb_drain.wait()

        plsc.subcore_barrier()
        pltpu.sync_copy(acc_sh.at[pl.ds(r0, RPD)], acc_o.at[cid, pl.ds(r0, RPD)])

    return k(e_pack, h, t_tab)


_BM = 1024


def _dense1(x_p, w1, m_t):
    def body(x_ref, w_ref, m_ref, h_ref, t_ref):
        h = jnp.dot(x_ref[...], w_ref[...], preferred_element_type=jnp.float32)
        h_ref[...] = h
        t_ref[...] = jnp.dot(h, m_ref[...], preferred_element_type=jnp.float32)

    return pl.pallas_call(
        body,
        grid=(NP // _BM,),
        in_specs=[
            pl.BlockSpec((_BM, 128), lambda i: (i, 0)),
            pl.BlockSpec((128, 128), lambda i: (0, 0)),
            pl.BlockSpec((128, 128), lambda i: (0, 0)),
        ],
        out_specs=[
            pl.BlockSpec((_BM, 128), lambda i: (i, 0)),
            pl.BlockSpec((_BM, 128), lambda i: (i, 0)),
        ],
        out_shape=[
            jax.ShapeDtypeStruct((NP, 128), jnp.float32),
            jax.ShapeDtypeStruct((NP, 128), jnp.float32),
        ],
    )(x_p, w1, m_t)


def _dense2(p0, p1, d0, d1, r_sel, b1, w2, m_t):
    def body(p0_ref, p1_ref, d0_ref, d1_ref, r_ref, b1_ref, w2_ref,
             m_ref, h_ref, t_ref):
        den = jnp.dot(d0_ref[...] + d1_ref[...], r_ref[...],
                      preferred_element_type=jnp.float32)
        agg = (p0_ref[...] + p1_ref[...]) / jnp.maximum(den, 1e-16)
        v = agg + b1_ref[...]
        x2 = jnp.where(v > 0.0, v, jnp.exp(v) - 1.0)
        h2 = jnp.dot(x2, w2_ref[...], preferred_element_type=jnp.float32)
        h_ref[...] = h2
        t_ref[...] = jnp.dot(h2, m_ref[...], preferred_element_type=jnp.float32)

    return pl.pallas_call(
        body,
        grid=(NP // _BM,),
        in_specs=[
            pl.BlockSpec((_BM, 128), lambda i: (i, 0)),
            pl.BlockSpec((_BM, 128), lambda i: (i, 0)),
            pl.BlockSpec((_BM, 16), lambda i: (i, 0)),
            pl.BlockSpec((_BM, 16), lambda i: (i, 0)),
            pl.BlockSpec((16, 128), lambda i: (0, 0)),
            pl.BlockSpec((1, 128), lambda i: (0, 0)),
            pl.BlockSpec((128, 128), lambda i: (0, 0)),
            pl.BlockSpec((128, 128), lambda i: (0, 0)),
        ],
        out_specs=[
            pl.BlockSpec((_BM, 128), lambda i: (i, 0)),
            pl.BlockSpec((_BM, 128), lambda i: (i, 0)),
        ],
        out_shape=[
            jax.ShapeDtypeStruct((NP, 128), jnp.float32),
            jax.ShapeDtypeStruct((NP, 128), jnp.float32),
        ],
    )(p0, p1, d0, d1, r_sel, b1, w2, m_t)


_BMC = 2000


def _dense3(p0, p1, d0, d1, r_sel, b2):
    def body(p0_ref, p1_ref, d0_ref, d1_ref, r_ref, b2_ref, o_ref):
        den = jnp.dot(d0_ref[...] + d1_ref[...], r_ref[...],
                      preferred_element_type=jnp.float32)
        o_ref[...] = ((p0_ref[...] + p1_ref[...])
                      / jnp.maximum(den, 1e-16) + b2_ref[...])

    return pl.pallas_call(
        body,
        grid=(N // _BMC,),
        in_specs=[
            pl.BlockSpec((_BMC, 128), lambda i: (i, 0)),
            pl.BlockSpec((_BMC, 128), lambda i: (i, 0)),
            pl.BlockSpec((_BMC, 16), lambda i: (i, 0)),
            pl.BlockSpec((_BMC, 16), lambda i: (i, 0)),
            pl.BlockSpec((16, 128), lambda i: (0, 0)),
            pl.BlockSpec((1, 128), lambda i: (0, 0)),
        ],
        out_specs=pl.BlockSpec((_BMC, 128), lambda i: (i, 0)),
        out_shape=jax.ShapeDtypeStruct((N, 128), jnp.float32),
    )(p0, p1, d0, d1, r_sel, b2)


def kernel(x, edge_index, W1, att_src1, att_dst1, b1, W2, att_src2, att_dst2, b2):
    f32 = jnp.float32
    loops = jnp.arange(N, dtype=jnp.int32)
    src = jnp.concatenate([edge_index[0], loops])
    dst = jnp.concatenate([edge_index[1], loops])
    pad = EP - (E + N)
    padv = jnp.full((pad,), N, jnp.int32)
    src_r = jnp.concatenate([src, padv]).reshape(NW, STEPS, 1, C)
    dst_r = jnp.concatenate([dst, padv]).reshape(NW, STEPS, 1, C)
    e_pack = jnp.concatenate([src_r, dst_r], axis=2).reshape(2 * EP)

    x_p = jnp.zeros((NP, IN), f32).at[:N].set(x)

    eye8 = jnp.eye(HEADS, dtype=f32)
    ms1 = (att_src1[:, :, None] * eye8[:, None, :]).reshape(HEADS * HID, HEADS)
    md1 = (att_dst1[:, :, None] * eye8[:, None, :]).reshape(HEADS * HID, HEADS)
    zpad = jnp.zeros((HEADS * HID, 96), f32)
    m1 = jnp.concatenate([ms1, ms1, md1, md1, zpad], axis=1)
    m2 = jnp.concatenate([jnp.tile(att_src2.T, (1, 16)),
                          jnp.tile(att_dst2.T, (1, 16)), zpad], axis=1)

    r1 = (jnp.arange(128)[None, :] // 16 == jnp.arange(16)[:, None]).astype(f32)
    r2 = (jnp.arange(16)[:, None] == 0).astype(f32) * jnp.ones((1, 128), f32)

    h1, t1 = _dense1(x_p, W1, m1)
    out1 = _edge_pass(e_pack, h1, t1)
    acc1 = out1[:, :NP]
    den1 = out1[:, NP:].reshape(NC, NP, 16)
    h2, t2 = _dense2(acc1[0], acc1[1], den1[0], den1[1], r1,
                     b1.reshape(1, 128), W2, m2)
    out2 = _edge_pass(e_pack, h2, t2)
    acc2 = out2[:, :NP]
    den2 = out2[:, NP:].reshape(NC, NP, 16)
    out = _dense3(acc2[0], acc2[1], den2[0], den2[1], r2, b2.reshape(1, 128))
    return out


# speedup vs baseline: 1.0628x; 1.0628x over previous
import functools

import jax
import jax.numpy as jnp
from jax import lax
from jax.experimental import pallas as pl
from jax.experimental.pallas import tpu as pltpu
from jax.experimental.pallas import tpu_sc as plsc

N = 10000
IN = 128
HID = 16
HEADS = 8
OUT = 128
E = 320000

NP = 10240
NPD = NP + NP // 8
NC = 2
NS = 16
NW = NC * NS
C = 16
STEPS = 646
EP = NW * C * STEPS
RPD = NPD // NS


def _edge_pass(e_pack, h, t_tab):
    mesh = plsc.VectorSubcoreMesh(core_axis_name="c", subcore_axis_name="s",
                                  num_cores=NC, num_subcores=NS)

    @functools.partial(
        pl.kernel,
        out_type=jax.ShapeDtypeStruct((NC, NPD, 128), jnp.float32),
        mesh=mesh,
        scratch_types=[
            pltpu.VMEM((STEPS * 2 * C,), jnp.int32),
            pltpu.VMEM((2 * C, 128), jnp.float32),
            pltpu.VMEM((2 * C, 128), jnp.float32),
            pltpu.VMEM((2 * C, 128), jnp.float32),
            pltpu.VMEM((2 * C, 128), jnp.float32),
            pltpu.VMEM((2 * C,), jnp.int32),
            pltpu.VMEM((2 * C,), jnp.int32),
            pltpu.VMEM_SHARED((NPD, 128), jnp.float32),
            pltpu.SemaphoreType.DMA,
            pltpu.SemaphoreType.DMA,
            pltpu.SemaphoreType.DMA,
            pltpu.SemaphoreType.DMA,
        ],
    )
    def k(e_h, h_h, t_h, acc_o, eidx_v, comb_a, comb_b, g_a, g_b,
          sci_a, sci_b, acc_sh, s0a, s1a, s0b, s1b):
        cid = lax.axis_index("c")
        sid = lax.axis_index("s")
        wid = cid * NS + sid
        r0 = sid * RPD

        pltpu.sync_copy(e_h.at[pl.ds(wid * STEPS * 2 * C, STEPS * 2 * C)],
                        eidx_v)

        for comb_v in (comb_a, comb_b):
            @pl.loop(0, 2 * C)
            def _(e, comb_v=comb_v):
                for c in range(8):
                    comb_v[e, pl.ds(16 * c, 16)] = jnp.zeros((16,),
                                                             jnp.float32)

        for p in range(RPD // 40):
            pltpu.sync_copy(comb_a.at[pl.ds(0, 40)],
                            acc_sh.at[pl.ds(r0 + p * 40, 40)])
        plsc.subcore_barrier()

        def launch(st, g_v, comb_v, s0, s1):
            ib = st * 2 * C
            cg = pltpu.async_copy(t_h.at[eidx_v.at[pl.ds(ib, 2 * C)]], g_v, s0)
            ch = pltpu.async_copy(h_h.at[eidx_v.at[pl.ds(ib, C)]],
                                  comb_v.at[pl.ds(0, C)], s1)
            return cg, ch

        def process(st, g_v, comb_v, scidx_v, cg, ch):
            ib = st * 2 * C
            cg.wait()
            ch.wait()

            @pl.loop(0, C // 16)
            def _(j):
                d16 = eidx_v[pl.ds(ib + C + 16 * j, 16)]
                scidx_v[pl.ds(16 * j, 16)] = d16
                scidx_v[pl.ds(C + 16 * j, 16)] = (
                    NP + lax.shift_right_logical(d16, 3))
                slot16 = (d16 & 7) * 16
                for i in range(16):
                    e = 16 * j + i
                    t = g_v[e, pl.ds(0, 16)] + g_v[C + e, pl.ds(16, 16)]
                    t = jnp.where(t >= 0.0, t, 0.2 * t)
                    w = jnp.exp(t)
                    comb_v[C + e, pl.ds(slot16[i], 16)] = w
                    for m in range(8):
                        comb_v[e, pl.ds(16 * m, 16)] = (
                            comb_v[e, pl.ds(16 * m, 16)] * w[m])

            pltpu.sync_copy(comb_v, acc_sh.at[scidx_v], add=True)

            @pl.loop(0, C // 16)
            def _(j):
                d16 = eidx_v[pl.ds(ib + C + 16 * j, 16)]
                slot16 = (d16 & 7) * 16
                for i in range(16):
                    comb_v[C + 16 * j + i, pl.ds(slot16[i], 16)] = (
                        jnp.zeros((16,), jnp.float32))

        prime = launch(0, g_a, comb_a, s0a, s1a)

        @pl.loop(0, STEPS, step=2)
        def _(st):
            cgb, chb = launch(st + 1, g_b, comb_b, s0b, s1b)
            process(st, g_a, comb_a, sci_a, *prime)

            @pl.when(st + 2 < STEPS)
            def _():
                launch(st + 2, g_a, comb_a, s0a, s1a)

            process(st + 1, g_b, comb_b, sci_b, cgb, chb)

        plsc.subcore_barrier()
        pltpu.sync_copy(acc_sh.at[pl.ds(r0, RPD)], acc_o.at[cid, pl.ds(r0, RPD)])

    return k(e_pack, h, t_tab)


_BM = 1024


def _dense1(x_p, w1, m_t):
    def body(x_ref, w_ref, m_ref, h_ref, t_ref):
        h = jnp.dot(x_ref[...], w_ref[...], preferred_element_type=jnp.float32)
        h_ref[...] = h
        t_ref[...] = jnp.dot(h, m_ref[...], preferred_element_type=jnp.float32)

    return pl.pallas_call(
        body,
        grid=(NP // _BM,),
        in_specs=[
            pl.BlockSpec((_BM, 128), lambda i: (i, 0)),
            pl.BlockSpec((128, 128), lambda i: (0, 0)),
            pl.BlockSpec((128, 128), lambda i: (0, 0)),
        ],
        out_specs=[
            pl.BlockSpec((_BM, 128), lambda i: (i, 0)),
            pl.BlockSpec((_BM, 128), lambda i: (i, 0)),
        ],
        out_shape=[
            jax.ShapeDtypeStruct((NP, 128), jnp.float32),
            jax.ShapeDtypeStruct((NP, 128), jnp.float32),
        ],
    )(x_p, w1, m_t)


def _dense2(p0, p1, d0, d1, r_sel, b1, w2, m_t):
    def body(p0_ref, p1_ref, d0_ref, d1_ref, r_ref, b1_ref, w2_ref,
             m_ref, h_ref, t_ref):
        den = jnp.dot(d0_ref[...] + d1_ref[...], r_ref[...],
                      preferred_element_type=jnp.float32)
        agg = (p0_ref[...] + p1_ref[...]) / jnp.maximum(den, 1e-16)
        v = agg + b1_ref[...]
        x2 = jnp.where(v > 0.0, v, jnp.exp(v) - 1.0)
        h2 = jnp.dot(x2, w2_ref[...], preferred_element_type=jnp.float32)
        h_ref[...] = h2
        t_ref[...] = jnp.dot(h2, m_ref[...], preferred_element_type=jnp.float32)

    return pl.pallas_call(
        body,
        grid=(NP // _BM,),
        in_specs=[
            pl.BlockSpec((_BM, 128), lambda i: (i, 0)),
            pl.BlockSpec((_BM, 128), lambda i: (i, 0)),
            pl.BlockSpec((_BM, 16), lambda i: (i, 0)),
            pl.BlockSpec((_BM, 16), lambda i: (i, 0)),
            pl.BlockSpec((16, 128), lambda i: (0, 0)),
            pl.BlockSpec((1, 128), lambda i: (0, 0)),
            pl.BlockSpec((128, 128), lambda i: (0, 0)),
            pl.BlockSpec((128, 128), lambda i: (0, 0)),
        ],
        out_specs=[
            pl.BlockSpec((_BM, 128), lambda i: (i, 0)),
            pl.BlockSpec((_BM, 128), lambda i: (i, 0)),
        ],
        out_shape=[
            jax.ShapeDtypeStruct((NP, 128), jnp.float32),
            jax.ShapeDtypeStruct((NP, 128), jnp.float32),
        ],
    )(p0, p1, d0, d1, r_sel, b1, w2, m_t)


_BMC = 2000


def _dense3(p0, p1, d0, d1, r_sel, b2):
    def body(p0_ref, p1_ref, d0_ref, d1_ref, r_ref, b2_ref, o_ref):
        den = jnp.dot(d0_ref[...] + d1_ref[...], r_ref[...],
                      preferred_element_type=jnp.float32)
        o_ref[...] = ((p0_ref[...] + p1_ref[...])
                      / jnp.maximum(den, 1e-16) + b2_ref[...])

    return pl.pallas_call(
        body,
        grid=(N // _BMC,),
        in_specs=[
            pl.BlockSpec((_BMC, 128), lambda i: (i, 0)),
            pl.BlockSpec((_BMC, 128), lambda i: (i, 0)),
            pl.BlockSpec((_BMC, 16), lambda i: (i, 0)),
            pl.BlockSpec((_BMC, 16), lambda i: (i, 0)),
            pl.BlockSpec((16, 128), lambda i: (0, 0)),
            pl.BlockSpec((1, 128), lambda i: (0, 0)),
        ],
        out_specs=pl.BlockSpec((_BMC, 128), lambda i: (i, 0)),
        out_shape=jax.ShapeDtypeStruct((N, 128), jnp.float32),
    )(p0, p1, d0, d1, r_sel, b2)


def kernel(x, edge_index, W1, att_src1, att_dst1, b1, W2, att_src2, att_dst2, b2):
    f32 = jnp.float32
    loops = jnp.arange(N, dtype=jnp.int32)
    src = jnp.concatenate([edge_index[0], loops])
    dst = jnp.concatenate([edge_index[1], loops])
    pad = EP - (E + N)
    padv = jnp.full((pad,), N, jnp.int32)
    src_r = jnp.concatenate([src, padv]).reshape(NW, STEPS, 1, C)
    dst_r = jnp.concatenate([dst, padv]).reshape(NW, STEPS, 1, C)
    e_pack = jnp.concatenate([src_r, dst_r], axis=2).reshape(2 * EP)

    x_p = jnp.zeros((NP, IN), f32).at[:N].set(x)

    eye8 = jnp.eye(HEADS, dtype=f32)
    ms1 = (att_src1[:, :, None] * eye8[:, None, :]).reshape(HEADS * HID, HEADS)
    md1 = (att_dst1[:, :, None] * eye8[:, None, :]).reshape(HEADS * HID, HEADS)
    zpad = jnp.zeros((HEADS * HID, 96), f32)
    m1 = jnp.concatenate([ms1, ms1, md1, md1, zpad], axis=1)
    m2 = jnp.concatenate([jnp.tile(att_src2.T, (1, 16)),
                          jnp.tile(att_dst2.T, (1, 16)), zpad], axis=1)

    r1 = (jnp.arange(128)[None, :] // 16 == jnp.arange(16)[:, None]).astype(f32)
    r2 = (jnp.arange(16)[:, None] == 0).astype(f32) * jnp.ones((1, 128), f32)

    h1, t1 = _dense1(x_p, W1, m1)
    out1 = _edge_pass(e_pack, h1, t1)
    acc1 = out1[:, :NP]
    den1 = out1[:, NP:].reshape(NC, NP, 16)
    h2, t2 = _dense2(acc1[0], acc1[1], den1[0], den1[1], r1,
                     b1.reshape(1, 128), W2, m2)
    out2 = _edge_pass(e_pack, h2, t2)
    acc2 = out2[:, :NP]
    den2 = out2[:, NP:].reshape(NC, NP, 16)
    out = _dense3(acc2[0], acc2[1], den2[0], den2[1], r2, b2.reshape(1, 128))
    return out
